# baseline (speedup 1.0000x reference)
import functools
import math

import jax
import jax.numpy as jnp
import numpy as np
from jax import lax
from jax.experimental import pallas as pl
from jax.experimental.pallas import tpu as pltpu
from jax.experimental.pallas import tpu_sc as plsc

N = 10000
E = 320000
D = 128
A = 16
B = 8
H = 8

NPAD = 10240
C = 16
EPT = E // 32
NCHUNK = EPT // C
R = 4

_INV_SQRT_B = 1.0 / math.sqrt(B)
_INV_SQRT_H = 1.0 / math.sqrt(H)
_INV_SQRT_D = 1.0 / math.sqrt(D)
_INV_SQRT_DA = 1.0 / math.sqrt(D * A)
_INV_SQRT_NEIGH = 1.0 / math.sqrt(32.0)


def _tc_pre_body(h_ref, x_ref, wl1_ref, wsc_ref, h1_ref, sc_ref):
    h = h_ref[...]
    x = x_ref[...]
    h1_ref[...] = jnp.dot(h, wl1_ref[...], preferred_element_type=jnp.float32) * _INV_SQRT_D
    acc = jnp.zeros(h.shape, jnp.float32)
    for v in range(A):
        acc += jnp.dot(h, wsc_ref[v], preferred_element_type=jnp.float32) * x[:, v:v + 1]
    sc_ref[...] = acc * _INV_SQRT_DA


def _tc_pre(h, x, w_lin1, wsc_t):
    nb = 1000
    grid = N // nb
    return pl.pallas_call(
        _tc_pre_body,
        grid=(grid,),
        in_specs=[
            pl.BlockSpec((nb, D), lambda i: (i, 0)),
            pl.BlockSpec((nb, A), lambda i: (i, 0)),
            pl.BlockSpec((D, D), lambda i: (0, 0)),
            pl.BlockSpec((A, D, D), lambda i: (0, 0, 0)),
        ],
        out_specs=[
            pl.BlockSpec((nb, D), lambda i: (i, 0)),
            pl.BlockSpec((nb, D), lambda i: (i, 0)),
        ],
        out_shape=[
            jax.ShapeDtypeStruct((N, D), jnp.float32),
            jax.ShapeDtypeStruct((N, D), jnp.float32),
        ],
    )(h, x, w_lin1, wsc_t)


def _tc_wp_body(el_ref, sh_ref, w1_ref, w2_ref, wp_ref):
    hid = jax.nn.silu(
        jnp.dot(el_ref[...], w1_ref[...], preferred_element_type=jnp.float32) * _INV_SQRT_B)
    wp = jnp.dot(hid, w2_ref[...], preferred_element_type=jnp.float32)
    wp_ref[...] = wp * (_INV_SQRT_H * _INV_SQRT_NEIGH) * sh_ref[...]


def _tc_wp(elen, esh, w_fc1, w_fc2):
    eb = 8000
    grid = E // eb
    return pl.pallas_call(
        _tc_wp_body,
        grid=(grid,),
        in_specs=[
            pl.BlockSpec((eb, B), lambda i: (i, 0)),
            pl.BlockSpec((eb, 1), lambda i: (i, 0)),
            pl.BlockSpec((B, H), lambda i: (0, 0)),
            pl.BlockSpec((H, D), lambda i: (0, 0)),
        ],
        out_specs=pl.BlockSpec((eb, D), lambda i: (i, 0)),
        out_shape=jax.ShapeDtypeStruct((E, D), jnp.float32),
    )(elen, esh, w_fc1, w_fc2)


def _sc_edge_body(h1_hbm, wp_hbm, src_hbm, dst_hbm, out_hbm,
                  sidx, didx, rows, wpb, sbuf, agg_sh, gsem, wsem, ssem,
                  zsem, d1sem, d2sem):
    cid = lax.axis_index("c")
    sid = lax.axis_index("s")
    wid = sid * 2 + cid
    ebase0 = wid * EPT

    pltpu.sync_copy(src_hbm.at[wid], sidx)
    pltpu.sync_copy(dst_hbm.at[wid], didx)

    def svec(ci):
        return sidx[ci >> 3, pl.ds((ci & 7) * 16, 16)]

    def dvec(ci):
        return didx[ci >> 3, pl.ds((ci & 7) * 16, 16)]

    zrow = jnp.zeros((16,), jnp.float32)

    def zbody(e, carry):
        for d in range(D // 16):
            sbuf[0][e, pl.ds(d * 16, 16)] = zrow
        return carry

    lax.fori_loop(0, C, zbody, 0)
    zbase = sid * (NPAD // 16)
    for k in range(40):
        pltpu.async_copy(sbuf[0], agg_sh.at[pl.ds(zbase + k * C, C)], zsem)
    for k in range(40):
        pltpu.make_async_copy(sbuf[0], agg_sh.at[pl.ds(zbase, C)], zsem).wait()
    plsc.subcore_barrier()

    def fire(ci, b):
        pltpu.async_copy(h1_hbm.at[svec(ci)], rows[b], gsem[b])
        pltpu.async_copy(wp_hbm.at[pl.ds(ebase0 + ci * C, C)], wpb[b], wsem[b])

    def drain(ci, b):
        pltpu.make_async_copy(h1_hbm.at[svec(ci)], rows[b], gsem[b]).wait()
        pltpu.make_async_copy(wp_hbm.at[pl.ds(0, C)], wpb[b], wsem[b]).wait()

    def mul(b):
        def mbody(e, c2):
            for d in range(D // 16):
                sl = pl.ds(d * 16, 16)
                sbuf[b][e, sl] = rows[b][e, sl] * wpb[b][e, sl]
            return c2
        lax.fori_loop(0, C, mbody, 0)

    def scat_wait(b):
        pltpu.make_async_copy(sbuf[b], agg_sh.at[dvec(0)], ssem[b]).wait()

    for b in range(R):
        fire(b, b)

    def tbody(t, carry):
        for b in range(R):
            ci = t * R + b
            drain(ci, b)

            @pl.when(t > 0)
            def _(_b=b):
                scat_wait(_b)

            mul(b)
            pltpu.async_copy(sbuf[b], agg_sh.at[dvec(ci)], ssem[b], add=True)

            @pl.when(ci + R < NCHUNK)
            def _(_b=b, _ci=ci):
                fire(_ci + R, _b)
        return carry

    lax.fori_loop(0, NCHUNK // R, tbody, 0)
    ci_tail = (NCHUNK // R) * R
    drain(ci_tail, 0)
    scat_wait(0)
    mul(0)
    pltpu.async_copy(sbuf[0], agg_sh.at[dvec(ci_tail)], ssem[0], add=True)
    for b in range(R):
        scat_wait(b)

    plsc.subcore_barrier()
    for g in range(10):
        for b in range(R):
            k = g * R + b
            if g > 0:
                pltpu.make_async_copy(sbuf[b], out_hbm.at[cid, pl.ds(zbase, C)],
                                      d2sem[b]).wait()
            pltpu.async_copy(agg_sh.at[pl.ds(zbase + k * C, C)], sbuf[b], d1sem[b])
        for b in range(R):
            k = g * R + b
            pltpu.make_async_copy(agg_sh.at[pl.ds(zbase, C)], sbuf[b], d1sem[b]).wait()
            pltpu.async_copy(sbuf[b], out_hbm.at[cid, pl.ds(zbase + k * C, C)], d2sem[b])
    for b in range(R):
        pltpu.make_async_copy(sbuf[b], out_hbm.at[cid, pl.ds(zbase, C)], d2sem[b]).wait()


def _sc_edge(h1, wp, src, dst):
    mesh = plsc.VectorSubcoreMesh(core_axis_name="c", subcore_axis_name="s")
    f = pl.kernel(
        _sc_edge_body,
        mesh=mesh,
        out_type=jax.ShapeDtypeStruct((2, NPAD, D), jnp.float32),
        scratch_types=[
            pltpu.VMEM((80, 128), jnp.int32),
            pltpu.VMEM((80, 128), jnp.int32),
            [pltpu.VMEM((C, D), jnp.float32)] * R,
            [pltpu.VMEM((C, D), jnp.float32)] * R,
            [pltpu.VMEM((C, D), jnp.float32)] * R,
            pltpu.VMEM_SHARED((NPAD, D), jnp.float32),
            [pltpu.SemaphoreType.DMA] * R,
            [pltpu.SemaphoreType.DMA] * R,
            [pltpu.SemaphoreType.DMA] * R,
            pltpu.SemaphoreType.DMA,
            [pltpu.SemaphoreType.DMA] * R,
            [pltpu.SemaphoreType.DMA] * R,
        ],
    )
    pad = jnp.zeros((32, 240), jnp.int32)
    srcp = jnp.concatenate([src.reshape(32, EPT), pad], axis=1).reshape(32, 80, 128)
    dstp = jnp.concatenate([dst.reshape(32, EPT), pad], axis=1).reshape(32, 80, 128)
    return f(h1, wp, srcp, dstp)


def _tc_post_body(agg_ref, sc_ref, wl2_ref, out_ref):
    a = agg_ref[0] + agg_ref[1]
    out_ref[...] = (
        jnp.dot(a, wl2_ref[...], preferred_element_type=jnp.float32) * _INV_SQRT_D
        + sc_ref[...])


def _tc_post(agg2, sc, w_lin2):
    nb = 1000
    grid = N // nb
    return pl.pallas_call(
        _tc_post_body,
        grid=(grid,),
        in_specs=[
            pl.BlockSpec((2, nb, D), lambda i: (0, i, 0)),
            pl.BlockSpec((nb, D), lambda i: (i, 0)),
            pl.BlockSpec((D, D), lambda i: (0, 0)),
        ],
        out_specs=pl.BlockSpec((nb, D), lambda i: (i, 0)),
        out_shape=jax.ShapeDtypeStruct((N, D), jnp.float32),
    )(agg2, sc, w_lin2)


def kernel(x, h, edge_length_embeddings, edge_sh, edge_index, W_lin1, W_fc1, W_fc2, W_lin2, W_sc):
    wsc_t = jnp.transpose(W_sc, (1, 0, 2))
    esh = edge_sh.reshape(E, 1)
    src = edge_index[1]
    dst = edge_index[0]

    h1h, sc = _tc_pre(h, x, W_lin1, wsc_t)
    wph = _tc_wp(edge_length_embeddings, esh, W_fc1, W_fc2)
    agg2 = _sc_edge(h1h, wph, src, dst)
    return _tc_post(agg2, sc, W_lin2)


# speedup vs baseline: 1.0323x; 1.0323x over previous
import functools
import math

import jax
import jax.numpy as jnp
import numpy as np
from jax import lax
from jax.experimental import pallas as pl
from jax.experimental.pallas import tpu as pltpu
from jax.experimental.pallas import tpu_sc as plsc

N = 10000
E = 320000
D = 128
A = 16
B = 8
H = 8

NPAD = 10240
C = 16
EPT = E // 32
NCHUNK = EPT // C
R = 4

_INV_SQRT_B = 1.0 / math.sqrt(B)
_INV_SQRT_H = 1.0 / math.sqrt(H)
_INV_SQRT_D = 1.0 / math.sqrt(D)
_INV_SQRT_DA = 1.0 / math.sqrt(D * A)
_INV_SQRT_NEIGH = 1.0 / math.sqrt(32.0)


def _tc_h1_body(h_ref, wl1_ref, h1_ref):
    h1_ref[...] = jnp.dot(h_ref[...], wl1_ref[...],
                          preferred_element_type=jnp.float32) * _INV_SQRT_D


def _tc_h1(h, w_lin1):
    nb = 1000
    grid = N // nb
    return pl.pallas_call(
        _tc_h1_body,
        grid=(grid,),
        in_specs=[
            pl.BlockSpec((nb, D), lambda i: (i, 0)),
            pl.BlockSpec((D, D), lambda i: (0, 0)),
        ],
        out_specs=pl.BlockSpec((nb, D), lambda i: (i, 0)),
        out_shape=jax.ShapeDtypeStruct((N, D), jnp.float32),
    )(h, w_lin1)


def _tc_sc_body(h_ref, x_ref, wsc_ref, sc_ref):
    h = h_ref[...]
    x = x_ref[...]
    acc = jnp.zeros(h.shape, jnp.float32)
    for v in range(A):
        acc += jnp.dot(h, wsc_ref[v], preferred_element_type=jnp.float32) * x[:, v:v + 1]
    sc_ref[...] = acc * _INV_SQRT_DA


def _tc_sc(h, x, wsc_t):
    nb = 1000
    grid = N // nb
    return pl.pallas_call(
        _tc_sc_body,
        grid=(grid,),
        in_specs=[
            pl.BlockSpec((nb, D), lambda i: (i, 0)),
            pl.BlockSpec((nb, A), lambda i: (i, 0)),
            pl.BlockSpec((A, D, D), lambda i: (0, 0, 0)),
        ],
        out_specs=pl.BlockSpec((nb, D), lambda i: (i, 0)),
        out_shape=jax.ShapeDtypeStruct((N, D), jnp.float32),
    )(h, x, wsc_t)


def _tc_wp_body(el_ref, sh_ref, w1_ref, w2_ref, wp_ref):
    hid = jax.nn.silu(
        jnp.dot(el_ref[...], w1_ref[...], preferred_element_type=jnp.float32) * _INV_SQRT_B)
    wp = jnp.dot(hid, w2_ref[...], preferred_element_type=jnp.float32)
    wp_ref[...] = wp * (_INV_SQRT_H * _INV_SQRT_NEIGH) * sh_ref[...]


def _tc_wp(elen, esh, w_fc1, w_fc2):
    eb = 8000
    grid = E // eb
    return pl.pallas_call(
        _tc_wp_body,
        grid=(grid,),
        in_specs=[
            pl.BlockSpec((eb, B), lambda i: (i, 0)),
            pl.BlockSpec((eb, 1), lambda i: (i, 0)),
            pl.BlockSpec((B, H), lambda i: (0, 0)),
            pl.BlockSpec((H, D), lambda i: (0, 0)),
        ],
        out_specs=pl.BlockSpec((eb, D), lambda i: (i, 0)),
        out_shape=jax.ShapeDtypeStruct((E, D), jnp.float32),
    )(elen, esh, w_fc1, w_fc2)


def _sc_edge_body(h1_hbm, wp_hbm, src_hbm, dst_hbm, out_hbm,
                  sidx, didx, rows, wpb, sbuf, agg_sh, gsem, wsem, ssem,
                  zsem, d1sem, d2sem):
    cid = lax.axis_index("c")
    sid = lax.axis_index("s")
    wid = sid * 2 + cid
    ebase0 = wid * EPT

    pltpu.sync_copy(src_hbm.at[wid], sidx)
    pltpu.sync_copy(dst_hbm.at[wid], didx)

    def svec(ci):
        return sidx[ci >> 3, pl.ds((ci & 7) * 16, 16)]

    def dvec(ci):
        return didx[ci >> 3, pl.ds((ci & 7) * 16, 16)]

    zrow = jnp.zeros((16,), jnp.float32)

    def zbody(e, carry):
        for d in range(D // 16):
            sbuf[0][e, pl.ds(d * 16, 16)] = zrow
        return carry

    lax.fori_loop(0, C, zbody, 0)
    zbase = sid * (NPAD // 16)
    for k in range(40):
        pltpu.async_copy(sbuf[0], agg_sh.at[pl.ds(zbase + k * C, C)], zsem)
    for k in range(40):
        pltpu.make_async_copy(sbuf[0], agg_sh.at[pl.ds(zbase, C)], zsem).wait()
    plsc.subcore_barrier()

    def fire(ci, b):
        pltpu.async_copy(h1_hbm.at[svec(ci)], rows[b], gsem[b])
        pltpu.async_copy(wp_hbm.at[pl.ds(ebase0 + ci * C, C)], wpb[b], wsem[b])

    def drain(ci, b):
        pltpu.make_async_copy(h1_hbm.at[svec(ci)], rows[b], gsem[b]).wait()
        pltpu.make_async_copy(wp_hbm.at[pl.ds(0, C)], wpb[b], wsem[b]).wait()

    def mul(b):
        def mbody(i, c2):
            for u in range(2):
                e = i * 2 + u
                for d in range(D // 16):
                    sl = pl.ds(d * 16, 16)
                    sbuf[b][e, sl] = rows[b][e, sl] * wpb[b][e, sl]
            return c2
        lax.fori_loop(0, C // 2, mbody, 0)

    def scat_wait(b):
        pltpu.make_async_copy(sbuf[b], agg_sh.at[dvec(0)], ssem[b]).wait()

    for b in range(R):
        fire(b, b)

    def tbody(t, carry):
        for b in range(R):
            ci = t * R + b
            drain(ci, b)

            @pl.when(t > 0)
            def _(_b=b):
                scat_wait(_b)

            mul(b)
            pltpu.async_copy(sbuf[b], agg_sh.at[dvec(ci)], ssem[b], add=True)

            @pl.when(ci + R < NCHUNK)
            def _(_b=b, _ci=ci):
                fire(_ci + R, _b)
        return carry

    lax.fori_loop(0, NCHUNK // R, tbody, 0)
    ci_tail = (NCHUNK // R) * R
    drain(ci_tail, 0)
    scat_wait(0)
    mul(0)
    pltpu.async_copy(sbuf[0], agg_sh.at[dvec(ci_tail)], ssem[0], add=True)
    for b in range(R):
        scat_wait(b)

    plsc.subcore_barrier()
    for g in range(10):
        for b in range(R):
            k = g * R + b
            if g > 0:
                pltpu.make_async_copy(sbuf[b], out_hbm.at[cid, pl.ds(zbase, C)],
                                      d2sem[b]).wait()
            pltpu.async_copy(agg_sh.at[pl.ds(zbase + k * C, C)], sbuf[b], d1sem[b])
        for b in range(R):
            k = g * R + b
            pltpu.make_async_copy(agg_sh.at[pl.ds(zbase, C)], sbuf[b], d1sem[b]).wait()
            pltpu.async_copy(sbuf[b], out_hbm.at[cid, pl.ds(zbase + k * C, C)], d2sem[b])
    for b in range(R):
        pltpu.make_async_copy(sbuf[b], out_hbm.at[cid, pl.ds(zbase, C)], d2sem[b]).wait()


def _sc_edge(h1, wp, src, dst):
    mesh = plsc.VectorSubcoreMesh(core_axis_name="c", subcore_axis_name="s")
    f = pl.kernel(
        _sc_edge_body,
        mesh=mesh,
        out_type=jax.ShapeDtypeStruct((2, NPAD, D), jnp.float32),
        scratch_types=[
            pltpu.VMEM((80, 128), jnp.int32),
            pltpu.VMEM((80, 128), jnp.int32),
            [pltpu.VMEM((C, D), jnp.float32)] * R,
            [pltpu.VMEM((C, D), jnp.float32)] * R,
            [pltpu.VMEM((C, D), jnp.float32)] * R,
            pltpu.VMEM_SHARED((NPAD, D), jnp.float32),
            [pltpu.SemaphoreType.DMA] * R,
            [pltpu.SemaphoreType.DMA] * R,
            [pltpu.SemaphoreType.DMA] * R,
            pltpu.SemaphoreType.DMA,
            [pltpu.SemaphoreType.DMA] * R,
            [pltpu.SemaphoreType.DMA] * R,
        ],
    )
    pad = jnp.zeros((32, 240), jnp.int32)
    srcp = jnp.concatenate([src.reshape(32, EPT), pad], axis=1).reshape(32, 80, 128)
    dstp = jnp.concatenate([dst.reshape(32, EPT), pad], axis=1).reshape(32, 80, 128)
    return f(h1, wp, srcp, dstp)


def _tc_post_body(agg_ref, sc_ref, wl2_ref, out_ref):
    a = agg_ref[0] + agg_ref[1]
    out_ref[...] = (
        jnp.dot(a, wl2_ref[...], preferred_element_type=jnp.float32) * _INV_SQRT_D
        + sc_ref[...])


def _tc_post(agg2, sc, w_lin2):
    nb = 1000
    grid = N // nb
    return pl.pallas_call(
        _tc_post_body,
        grid=(grid,),
        in_specs=[
            pl.BlockSpec((2, nb, D), lambda i: (0, i, 0)),
            pl.BlockSpec((nb, D), lambda i: (i, 0)),
            pl.BlockSpec((D, D), lambda i: (0, 0)),
        ],
        out_specs=pl.BlockSpec((nb, D), lambda i: (i, 0)),
        out_shape=jax.ShapeDtypeStruct((N, D), jnp.float32),
    )(agg2, sc, w_lin2)


def kernel(x, h, edge_length_embeddings, edge_sh, edge_index, W_lin1, W_fc1, W_fc2, W_lin2, W_sc):
    wsc_t = jnp.transpose(W_sc, (1, 0, 2))
    esh = edge_sh.reshape(E, 1)
    src = edge_index[1]
    dst = edge_index[0]

    h1 = _tc_h1(h, W_lin1)
    wp = _tc_wp(edge_length_embeddings, esh, W_fc1, W_fc2)
    agg2 = _sc_edge(h1, wp, src, dst)
    sc = _tc_sc(h, x, wsc_t)
    return _tc_post(agg2, sc, W_lin2)


# speedup vs baseline: 1.0324x; 1.0001x over previous
import math

import jax
import jax.numpy as jnp
from jax import lax
from jax.experimental import pallas as pl
from jax.experimental.pallas import tpu as pltpu
from jax.experimental.pallas import tpu_sc as plsc

N = 10000
E = 320000
D = 128
A = 16
B = 8
H = 8

NPAD = 10240
C = 16
EPT = E // 32
NCHUNK = EPT // C
R = 4

_INV_SQRT_B = 1.0 / math.sqrt(B)
_INV_SQRT_H = 1.0 / math.sqrt(H)
_INV_SQRT_D = 1.0 / math.sqrt(D)
_INV_SQRT_DA = 1.0 / math.sqrt(D * A)
_INV_SQRT_NEIGH = 1.0 / math.sqrt(32.0)


def _tc_h1_body(h_ref, wl1_ref, h1_ref):
    h1_ref[...] = jnp.dot(h_ref[...], wl1_ref[...],
                          preferred_element_type=jnp.float32) * _INV_SQRT_D


def _tc_h1(h, w_lin1):
    nb = 1000
    grid = N // nb
    return pl.pallas_call(
        _tc_h1_body,
        grid=(grid,),
        in_specs=[
            pl.BlockSpec((nb, D), lambda i: (i, 0)),
            pl.BlockSpec((D, D), lambda i: (0, 0)),
        ],
        out_specs=pl.BlockSpec((nb, D), lambda i: (i, 0)),
        out_shape=jax.ShapeDtypeStruct((N, D), jnp.float32),
    )(h, w_lin1)


def _tc_sc_body(h_ref, x_ref, wsc_ref, sc_ref):
    h = h_ref[...]
    x = x_ref[...]
    acc = jnp.zeros(h.shape, jnp.float32)
    for v in range(A):
        acc += jnp.dot(h, wsc_ref[v], preferred_element_type=jnp.float32) * x[:, v:v + 1]
    sc_ref[...] = acc * _INV_SQRT_DA


def _tc_sc(h, x, wsc_t):
    nb = 1000
    grid = N // nb
    return pl.pallas_call(
        _tc_sc_body,
        grid=(grid,),
        in_specs=[
            pl.BlockSpec((nb, D), lambda i: (i, 0)),
            pl.BlockSpec((nb, A), lambda i: (i, 0)),
            pl.BlockSpec((A, D, D), lambda i: (0, 0, 0)),
        ],
        out_specs=pl.BlockSpec((nb, D), lambda i: (i, 0)),
        out_shape=jax.ShapeDtypeStruct((N, D), jnp.float32),
    )(h, x, wsc_t)


def _tc_wp_body(el_ref, sh_ref, w1_ref, w2_ref, wp_ref):
    hid = jax.nn.silu(
        jnp.dot(el_ref[...], w1_ref[...], preferred_element_type=jnp.float32) * _INV_SQRT_B)
    wp = jnp.dot(hid, w2_ref[...], preferred_element_type=jnp.float32)
    wp_ref[...] = wp * (_INV_SQRT_H * _INV_SQRT_NEIGH) * sh_ref[...]


def _tc_wp(elen, esh, w_fc1, w_fc2):
    eb = 8000
    grid = E // eb
    return pl.pallas_call(
        _tc_wp_body,
        grid=(grid,),
        in_specs=[
            pl.BlockSpec((eb, B), lambda i: (i, 0)),
            pl.BlockSpec((eb, 1), lambda i: (i, 0)),
            pl.BlockSpec((B, H), lambda i: (0, 0)),
            pl.BlockSpec((H, D), lambda i: (0, 0)),
        ],
        out_specs=pl.BlockSpec((eb, D), lambda i: (i, 0)),
        out_shape=jax.ShapeDtypeStruct((E, D), jnp.float32),
    )(elen, esh, w_fc1, w_fc2)


def _sc_edge_body(h1_hbm, wp_hbm, src_hbm, dst_hbm, out_hbm,
                  sidx, didx, rows, wpb, sbuf, agg_sh, gsem, wsem, ssem,
                  zsem, d1sem, d2sem):
    cid = lax.axis_index("c")
    sid = lax.axis_index("s")
    wid = sid * 2 + cid
    ebase0 = wid * EPT

    pltpu.sync_copy(src_hbm.at[wid], sidx)
    pltpu.sync_copy(dst_hbm.at[wid], didx)

    def svec(ci):
        return sidx[ci >> 3, pl.ds((ci & 7) * 16, 16)]

    def dvec(ci):
        return didx[ci >> 3, pl.ds((ci & 7) * 16, 16)]

    zrow = jnp.zeros((16,), jnp.float32)

    def zbody(e, carry):
        for d in range(D // 16):
            sbuf[0][e, pl.ds(d * 16, 16)] = zrow
        return carry

    lax.fori_loop(0, C, zbody, 0)
    zbase = sid * (NPAD // 16)
    for k in range(40):
        pltpu.async_copy(sbuf[0], agg_sh.at[pl.ds(zbase + k * C, C)], zsem)
    for k in range(40):
        pltpu.make_async_copy(sbuf[0], agg_sh.at[pl.ds(zbase, C)], zsem).wait()
    plsc.subcore_barrier()

    def fire(ci, b):
        pltpu.async_copy(h1_hbm.at[svec(ci)], rows[b], gsem[b])
        pltpu.async_copy(wp_hbm.at[pl.ds(ebase0 + ci * C, C)], wpb[b], wsem[b])

    def drain(ci, b):
        pltpu.make_async_copy(h1_hbm.at[svec(ci)], rows[b], gsem[b]).wait()
        pltpu.make_async_copy(wp_hbm.at[pl.ds(0, C)], wpb[b], wsem[b]).wait()

    def mul(b):
        def mbody(i, c2):
            for u in range(2):
                e = i * 2 + u
                for d in range(D // 16):
                    sl = pl.ds(d * 16, 16)
                    sbuf[b][e, sl] = rows[b][e, sl] * wpb[b][e, sl]
            return c2
        lax.fori_loop(0, C // 2, mbody, 0)

    def scat_wait(b):
        pltpu.make_async_copy(sbuf[b], agg_sh.at[dvec(0)], ssem[b]).wait()

    for b in range(R):
        fire(b, b)

    def tbody(t, carry):
        for b in range(R):
            ci = t * R + b
            drain(ci, b)

            @pl.when(t > 0)
            def _(_b=b):
                scat_wait(_b)

            mul(b)
            pltpu.async_copy(sbuf[b], agg_sh.at[dvec(ci)], ssem[b], add=True)

            @pl.when(ci + R < NCHUNK)
            def _(_b=b, _ci=ci):
                fire(_ci + R, _b)
        return carry

    lax.fori_loop(0, NCHUNK // R, tbody, 0)
    ci_tail = (NCHUNK // R) * R
    drain(ci_tail, 0)
    scat_wait(0)
    mul(0)
    pltpu.async_copy(sbuf[0], agg_sh.at[dvec(ci_tail)], ssem[0], add=True)
    for b in range(R):
        scat_wait(b)

    plsc.subcore_barrier()
    for g in range(10):
        for b in range(R):
            k = g * R + b
            if g > 0:
                pltpu.make_async_copy(sbuf[b], out_hbm.at[cid, pl.ds(zbase, C)],
                                      d2sem[b]).wait()
            pltpu.async_copy(agg_sh.at[pl.ds(zbase + k * C, C)], sbuf[b], d1sem[b])
        for b in range(R):
            k = g * R + b
            pltpu.make_async_copy(agg_sh.at[pl.ds(zbase, C)], sbuf[b], d1sem[b]).wait()
            pltpu.async_copy(sbuf[b], out_hbm.at[cid, pl.ds(zbase + k * C, C)], d2sem[b])
    for b in range(R):
        pltpu.make_async_copy(sbuf[b], out_hbm.at[cid, pl.ds(zbase, C)], d2sem[b]).wait()


def _sc_edge(h1, wp, src, dst):
    mesh = plsc.VectorSubcoreMesh(core_axis_name="c", subcore_axis_name="s")
    f = pl.kernel(
        _sc_edge_body,
        mesh=mesh,
        out_type=jax.ShapeDtypeStruct((2, NPAD, D), jnp.float32),
        scratch_types=[
            pltpu.VMEM((80, 128), jnp.int32),
            pltpu.VMEM((80, 128), jnp.int32),
            [pltpu.VMEM((C, D), jnp.float32)] * R,
            [pltpu.VMEM((C, D), jnp.float32)] * R,
            [pltpu.VMEM((C, D), jnp.float32)] * R,
            pltpu.VMEM_SHARED((NPAD, D), jnp.float32),
            [pltpu.SemaphoreType.DMA] * R,
            [pltpu.SemaphoreType.DMA] * R,
            [pltpu.SemaphoreType.DMA] * R,
            pltpu.SemaphoreType.DMA,
            [pltpu.SemaphoreType.DMA] * R,
            [pltpu.SemaphoreType.DMA] * R,
        ],
    )
    pad = jnp.zeros((32, 240), jnp.int32)
    srcp = jnp.concatenate([src.reshape(32, EPT), pad], axis=1).reshape(32, 80, 128)
    dstp = jnp.concatenate([dst.reshape(32, EPT), pad], axis=1).reshape(32, 80, 128)
    return f(h1, wp, srcp, dstp)


def _tc_post_body(agg_ref, sc_ref, wl2_ref, out_ref):
    a = agg_ref[0] + agg_ref[1]
    out_ref[...] = (
        jnp.dot(a, wl2_ref[...], preferred_element_type=jnp.float32) * _INV_SQRT_D
        + sc_ref[...])


def _tc_post(agg2, sc, w_lin2):
    nb = 1000
    grid = N // nb
    return pl.pallas_call(
        _tc_post_body,
        grid=(grid,),
        in_specs=[
            pl.BlockSpec((2, nb, D), lambda i: (0, i, 0)),
            pl.BlockSpec((nb, D), lambda i: (i, 0)),
            pl.BlockSpec((D, D), lambda i: (0, 0)),
        ],
        out_specs=pl.BlockSpec((nb, D), lambda i: (i, 0)),
        out_shape=jax.ShapeDtypeStruct((N, D), jnp.float32),
    )(agg2, sc, w_lin2)


def kernel(x, h, edge_length_embeddings, edge_sh, edge_index, W_lin1, W_fc1, W_fc2, W_lin2, W_sc):
    wsc_t = jnp.transpose(W_sc, (1, 0, 2))
    esh = edge_sh.reshape(E, 1)
    src = edge_index[1]
    dst = edge_index[0]

    h1 = _tc_h1(h, W_lin1)
    wp = _tc_wp(edge_length_embeddings, esh, W_fc1, W_fc2)
    agg2 = _sc_edge(h1, wp, src, dst)
    sc = _tc_sc(h, x, wsc_t)
    return _tc_post(agg2, sc, W_lin2)
